# baseline (speedup 1.0000x reference)
import jax
import jax.numpy as jnp
from jax.experimental import pallas as pl
from jax.experimental.pallas import tpu as pltpu

N = 16
E = 64
H = 128

_SHAPES = []
for _l in range(4):
    _SHAPES += [(45 if _l == 0 else H, H), (1, H), (H, 1), (H, 1), (1, 1)]
_SHAPES += [(H, H), (1, H)]
for _l in range(3):
    _SHAPES += [(256, 256), (1, 256), (1, 256), (1, 256)]
_SHAPES += [(256, 256), (1, 256)]
_NP = len(_SHAPES)


def _net_kernel(*refs):
    f32 = jnp.float32
    x_ref, ei_ref = refs[0], refs[1]
    hbm = refs[2:2 + _NP]
    out_ref = refs[2 + _NP]
    vmem = refs[3 + _NP:3 + 2 * _NP]
    sem = refs[3 + 2 * _NP]

    copies = [pltpu.make_async_copy(hbm[i], vmem[i], sem.at[i])
              for i in range(_NP)]
    for c in copies:
        c.start()

    state = {'i': 0}

    def nxt():
        i = state['i']
        copies[i].wait()
        state['i'] = i + 1
        return vmem[i][:, :]

    def dotT(a, b):
        return jax.lax.dot_general(a, b, (((0,), (0,)), ((), ())),
                                   preferred_element_type=f32)

    def mm(a, b):
        return jax.lax.dot_general(a, b, (((1,), (0,)), ((), ())),
                                   preferred_element_type=f32)

    srcT = ei_ref[0:1, :]
    dstT = ei_ref[1:2, :]
    rowN = jax.lax.broadcasted_iota(jnp.int32, (N, E), 0)
    ST = (srcT == rowN).astype(f32)
    DT = (dstT == rowN).astype(f32)
    mask = jnp.ones((1, E), dtype=f32)

    row_i = jax.lax.broadcasted_iota(jnp.int32, (N, N), 0)
    col_i = jax.lax.broadcasted_iota(jnp.int32, (N, N), 1)
    eye = (row_i == col_i).astype(f32)
    colf = col_i.astype(f32)
    valid_col = jax.lax.broadcasted_iota(jnp.int32, (N, 1), 0)

    x = x_ref[:, :]

    n_cur = N
    for l in range(4):
        W = nxt()
        b = nxt()
        Wrel = nxt()
        Wroot = nxt()
        brel = nxt()

        xw = mm(x, W)
        deg = jnp.sum(DT * mask, axis=1, keepdims=True) + 1.0
        dinv = 1.0 / jnp.sqrt(deg)
        norm = mask * dotT(dinv, ST) * dotT(dinv, DT)
        gath = dotT(ST, xw)
        aggc = mm(DT * norm, gath)
        x = jax.nn.relu(aggc + (1.0 / deg) * xw + b)

        agg2 = mm(DT * mask, dotT(ST, x))
        raw = mm(agg2, Wrel) + brel + mm(x, Wroot)
        score = jnp.tanh(raw)
        score = jnp.where(valid_col < n_cur, score, -2.0)

        k = (n_cur + 1) // 2
        s_row = dotT(score, eye)
        s_cb = jax.lax.broadcast_in_dim(score, (N, N), (0, 1))
        s_rb = jax.lax.broadcast_in_dim(s_row, (N, N), (0, 1))
        beats = (s_rb > s_cb) | ((s_rb == s_cb) & (col_i < row_i))
        rank = jnp.sum(beats.astype(f32), axis=1, keepdims=True)
        PT = ((rank == colf) & (colf < float(k))).astype(f32)

        sel_score = dotT(PT, score)
        x = dotT(PT, x) * sel_score
        ST = dotT(PT, ST)
        DT = dotT(PT, DT)
        mask = (mask * jnp.sum(ST, axis=0, keepdims=True)
                     * jnp.sum(DT, axis=0, keepdims=True))
        n_cur = k

    lin_W = nxt()
    lin_b = nxt()
    out2 = mm(x[0:1, :], lin_W) + lin_b
    m = jnp.max(out2, axis=1, keepdims=True)
    z = out2 - m
    out2 = z - jnp.log(jnp.sum(jnp.exp(z), axis=1, keepdims=True))

    h = jnp.concatenate([jnp.zeros((1, H), dtype=f32), out2], axis=1)

    for l in range(3):
        fcW = nxt()
        fcb = nxt()
        lnw = nxt()
        lnb = nxt()
        h = mm(h, fcW) + fcb
        mu = jnp.mean(h, axis=1, keepdims=True)
        var = jnp.mean((h - mu) ** 2, axis=1, keepdims=True)
        h = (h - mu) / jnp.sqrt(var + 1e-5) * lnw + lnb
        h = jax.nn.relu(h)

    fc3W = nxt()
    fc3b = nxt()
    out_ref[:, :] = mm(h, fc3W) + fc3b


def kernel(sp_x, sp_edge_index, params):
    f32 = jnp.float32
    p = params

    ei = jnp.zeros((8, E), dtype=jnp.int32).at[:2, :].set(
        sp_edge_index.astype(jnp.int32))

    args = []
    for l in range(4):
        args += [
            p['conv%d_W' % l],
            p['conv%d_b' % l].reshape(1, H),
            p['pool%d_Wrel' % l],
            p['pool%d_Wroot' % l],
            p['pool%d_brel' % l].reshape(1, 1),
        ]
    args += [p['lin_W'], p['lin_b'].reshape(1, H)]
    for l in range(3):
        args += [
            p['fc%d_W' % l],
            p['fc%d_b' % l].reshape(1, 256),
            p['ln%d_w' % l].reshape(1, 256),
            p['ln%d_b' % l].reshape(1, 256),
        ]
    args += [p['fc3_W'], p['fc3_b'].reshape(1, 256)]

    out = pl.pallas_call(
        _net_kernel,
        out_shape=jax.ShapeDtypeStruct((1, 256), f32),
        in_specs=[pl.BlockSpec(memory_space=pltpu.VMEM)] * 2
                 + [pl.BlockSpec(memory_space=pltpu.MemorySpace.HBM)] * _NP,
        out_specs=pl.BlockSpec(memory_space=pltpu.VMEM),
        scratch_shapes=[pltpu.VMEM(s, f32) for s in _SHAPES]
                       + [pltpu.SemaphoreType.DMA((_NP,))],
    )(sp_x, ei, *args)
    return out.reshape(-1)
